# baseline (speedup 1.0000x reference)
import functools

import jax
import jax.numpy as jnp
from jax import lax
from jax.experimental import pallas as pl
from jax.experimental.pallas import tpu as pltpu
from jax.experimental.pallas import tpu_sc as plsc

_NC = 2
_NS = 16
_NW = _NC * _NS
_L = 16

_BAND = 128
_WIDE = 128


@functools.lru_cache(maxsize=None)
def _make_gather(T, BATCH, V, D):
    mesh = plsc.VectorSubcoreMesh(core_axis_name="c", subcore_axis_name="s")

    @functools.partial(
        pl.kernel,
        out_type=jax.ShapeDtypeStruct((T, D, BATCH), jnp.float32),
        mesh=mesh,
        scratch_types=[
            pltpu.VMEM((8, _BAND), jnp.int32),
            pltpu.VMEM((2, _BAND), jnp.int32),
            pltpu.VMEM((2, _BAND, _WIDE), jnp.float32),
            pltpu.VMEM((D, _BAND), jnp.float32),
            pltpu.SemaphoreType.DMA,
        ],
        compiler_params=pltpu.CompilerParams(
            use_tc_tiling_on_sc=True, needs_layout_passes=False),
    )
    def k(table_hbm, idx_hbm, out_hbm, idx_v, pidx_v, rows_v, trans_v, gsem):
        wid = lax.axis_index("s") * _NC + lax.axis_index("c")
        bcol = pl.multiple_of(wid * _BAND, _BAND)

        iota = lax.iota(jnp.int32, _L)

        def load_octet(t):
            row = pl.multiple_of((t // 8) * 8, 8)
            pltpu.sync_copy(idx_hbm.at[pl.ds(row, 8), pl.ds(bcol, _BAND)],
                            idx_v)

        def stage_unit(u, buf):
            ts = lax.rem(u, 8)
            for g in range(_BAND // _L):
                pidx_v[buf, pl.ds(g * _L, _L)] = idx_v[ts, pl.ds(g * _L, _L)]

        def gather_start(buf):
            pltpu.async_copy(table_hbm.at[pidx_v.at[buf]], rows_v.at[buf],
                             gsem)

        def gather_wait(buf):
            pltpu.make_async_copy(table_hbm.at[pidx_v.at[buf]],
                                  rows_v.at[buf], gsem).wait()

        def transpose_store(t, buf):
            rv = rows_v.at[buf]
            for g in range(_BAND // _L):
                jb = g * _L
                rowv = iota + jb
                cv = iota - iota
                for d in range(D):
                    trans_v[d, pl.ds(jb, _L)] = plsc.load_gather(rv,
                                                                 [rowv, cv])
                    cv = cv + 1
            pltpu.sync_copy(trans_v, out_hbm.at[t, :, pl.ds(bcol, _BAND)])

        load_octet(0)
        stage_unit(0, 0)
        gather_start(0)

        def step(u, buf):
            nbuf = 1 - buf

            def launch_next():
                pl.when(lax.rem(u, 8) == 7)(lambda: load_octet(u + 1))
                stage_unit(u + 1, nbuf)
                gather_start(nbuf)

            pl.when(u + 1 < T)(launch_next)

            gather_wait(buf)
            transpose_store(u, buf)

        def pair_body(p, carry):
            step(2 * p, 0)
            step(2 * p + 1, 1)
            return carry

        lax.fori_loop(0, T // 2, pair_body, 0)

    return k


def kernel(words, weight):
    BATCH, T = words.shape
    V, D = weight.shape
    wpad = jnp.pad(weight, ((0, 0), (0, _WIDE - D)))
    idx_t = words.T.astype(jnp.int32)
    out3 = _make_gather(T, BATCH, V, D)(wpad, idx_t)
    return jnp.transpose(out3, (2, 0, 1))


# speedup vs baseline: 1.5564x; 1.5564x over previous
import functools

import jax
import jax.numpy as jnp
from jax import lax
from jax.experimental import pallas as pl
from jax.experimental.pallas import tpu as pltpu
from jax.experimental.pallas import tpu_sc as plsc

_NC = 2
_NS = 16
_NW = _NC * _NS
_L = 16

_BAND = 128
_WIDE = 128


@functools.lru_cache(maxsize=None)
def _make_gather(T, BATCH, V, D):
    mesh = plsc.VectorSubcoreMesh(core_axis_name="c", subcore_axis_name="s")

    @functools.partial(
        pl.kernel,
        out_type=jax.ShapeDtypeStruct((T, D, BATCH), jnp.float32),
        mesh=mesh,
        scratch_types=[
            pltpu.VMEM((8, _BAND), jnp.int32),
            pltpu.VMEM((2, _BAND), jnp.int32),
            pltpu.VMEM((2, _BAND, _WIDE), jnp.float32),
            pltpu.VMEM((D, _BAND), jnp.float32),
            pltpu.SemaphoreType.DMA,
        ],
        compiler_params=pltpu.CompilerParams(
            use_tc_tiling_on_sc=True, needs_layout_passes=False),
    )
    def k(table_hbm, idx_hbm, out_hbm, idx_v, pidx_v, rows_v, trans_v, gsem):
        wid = lax.axis_index("s") * _NC + lax.axis_index("c")
        bcol = pl.multiple_of(wid * _BAND, _BAND)

        iota = lax.iota(jnp.int32, _L)

        def load_octet(t):
            row = pl.multiple_of((t // 8) * 8, 8)
            pltpu.sync_copy(idx_hbm.at[pl.ds(row, 8), pl.ds(bcol, _BAND)],
                            idx_v)

        def stage_unit(u, buf):
            ts = lax.rem(u, 8)
            for g in range(_BAND // _L):
                pidx_v[buf, pl.ds(g * _L, _L)] = idx_v[ts, pl.ds(g * _L, _L)]

        def gather_start(buf):
            pltpu.async_copy(table_hbm.at[pidx_v.at[buf]], rows_v.at[buf],
                             gsem)

        def gather_wait(buf):
            pltpu.make_async_copy(table_hbm.at[pidx_v.at[buf]],
                                  rows_v.at[buf], gsem).wait()

        zero = iota - iota

        def transpose_store(t, buf):
            rv = rows_v.at[buf]
            for g in range(_BAND // _L):
                jb = g * _L
                rowv = iota + jb

                @plsc.parallel_loop(0, D, 1, unroll=8)
                def _(d):
                    cv = zero + d
                    trans_v[d, pl.ds(jb, _L)] = plsc.load_gather(rv,
                                                                 [rowv, cv])

            pltpu.sync_copy(trans_v, out_hbm.at[t, :, pl.ds(bcol, _BAND)])

        load_octet(0)
        stage_unit(0, 0)
        gather_start(0)

        def step(u, buf):
            nbuf = 1 - buf

            def launch_next():
                pl.when(lax.rem(u, 8) == 7)(lambda: load_octet(u + 1))
                stage_unit(u + 1, nbuf)
                gather_start(nbuf)

            pl.when(u + 1 < T)(launch_next)

            gather_wait(buf)
            transpose_store(u, buf)

        def pair_body(p, carry):
            step(2 * p, 0)
            step(2 * p + 1, 1)
            return carry

        lax.fori_loop(0, T // 2, pair_body, 0)

    return k


def kernel(words, weight):
    BATCH, T = words.shape
    V, D = weight.shape
    wpad = jnp.pad(weight, ((0, 0), (0, _WIDE - D)))
    idx_t = words.T.astype(jnp.int32)
    out3 = _make_gather(T, BATCH, V, D)(wpad, idx_t)
    return jnp.transpose(out3, (2, 0, 1))


# speedup vs baseline: 1.6414x; 1.0546x over previous
import functools

import jax
import jax.numpy as jnp
from jax import lax
from jax.experimental import pallas as pl
from jax.experimental.pallas import tpu as pltpu
from jax.experimental.pallas import tpu_sc as plsc

_NC = 2
_NS = 16
_NW = _NC * _NS
_L = 16

_BAND = 128
_WIDE = 128


@functools.lru_cache(maxsize=None)
def _make_gather(T, BATCH, V, D):
    mesh = plsc.VectorSubcoreMesh(core_axis_name="c", subcore_axis_name="s")

    @functools.partial(
        pl.kernel,
        out_type=jax.ShapeDtypeStruct((T, D, BATCH), jnp.float32),
        mesh=mesh,
        scratch_types=[
            pltpu.VMEM((2, 8, _BAND), jnp.int32),
            pltpu.VMEM((4, _BAND, _WIDE), jnp.float32),
            pltpu.VMEM((2, D, _BAND), jnp.float32),
            pltpu.SemaphoreType.DMA,
            pltpu.SemaphoreType.DMA,
            pltpu.SemaphoreType.DMA,
            pltpu.SemaphoreType.DMA,
            pltpu.SemaphoreType.DMA,
            pltpu.SemaphoreType.DMA,
        ],
        compiler_params=pltpu.CompilerParams(
            use_tc_tiling_on_sc=True, needs_layout_passes=False),
    )
    def k(table_hbm, idx_hbm, out_hbm, idx_v, rows_v, trans_v,
          gsem0, gsem1, gsem2, gsem3, ssem0, ssem1):
        wid = lax.axis_index("s") * _NC + lax.axis_index("c")
        bcol = pl.multiple_of(wid * _BAND, _BAND)

        iota = lax.iota(jnp.int32, _L)
        zero = iota - iota
        gsems = (gsem0, gsem1, gsem2, gsem3)
        ssems = (ssem0, ssem1)

        def load_octet(u):
            row = pl.multiple_of((u // 8) * 8, 8)
            ob = lax.rem(u // 8, 2)
            pltpu.sync_copy(idx_hbm.at[pl.ds(row, 8), pl.ds(bcol, _BAND)],
                            idx_v.at[ob])

        def idx_ref(u):
            ob = lax.rem(u // 8, 2)
            ts = lax.rem(u, 8)
            return idx_v.at[ob, ts]

        def gather_start(u, gbuf):
            pltpu.async_copy(table_hbm.at[idx_ref(u)], rows_v.at[gbuf],
                             gsems[gbuf])

        def gather_wait(u, gbuf):
            pltpu.make_async_copy(table_hbm.at[idx_ref(u)], rows_v.at[gbuf],
                                  gsems[gbuf]).wait()

        def out_slice(u):
            return out_hbm.at[u, :, pl.ds(bcol, _BAND)]

        def store_start(u, tbuf, par):
            pltpu.async_copy(trans_v.at[tbuf], out_slice(u), ssems[par])

        def store_wait(u, tbuf, par):
            pltpu.make_async_copy(trans_v.at[tbuf], out_slice(0),
                                  ssems[par]).wait()

        def transpose(tbuf, gbuf):
            rv = rows_v.at[gbuf]
            for g in range(_BAND // _L):
                jb = g * _L
                rowv = iota + jb

                @plsc.parallel_loop(0, D, 1, unroll=8)
                def _(d):
                    cv = zero + d
                    trans_v[tbuf, d, pl.ds(jb, _L)] = plsc.load_gather(
                        rv, [rowv, cv])

        load_octet(0)
        gather_start(0, 0)
        gather_start(1, 1)

        def step(u, s):
            par = s % 2
            nxt = u + 2

            def launch_next():
                pl.when(lax.rem(nxt, 8) == 0)(lambda: load_octet(nxt))
                gather_start(nxt, (s + 2) % 4)

            pl.when(nxt < T)(launch_next)
            pl.when(u >= 2)(lambda: store_wait(u - 2, par, par))
            gather_wait(u, s)
            transpose(par, s)
            store_start(u, par, par)

        def quad_body(p, carry):
            for s in range(4):
                step(4 * p + s, s)
            return carry

        lax.fori_loop(0, T // 4, quad_body, 0)
        store_wait(T - 2, 0, 0)
        store_wait(T - 1, 1, 1)

    return k


def kernel(words, weight):
    BATCH, T = words.shape
    V, D = weight.shape
    wpad = jnp.pad(weight, ((0, 0), (0, _WIDE - D)))
    idx_t = words.T.astype(jnp.int32)
    out3 = _make_gather(T, BATCH, V, D)(wpad, idx_t)
    return jnp.transpose(out3, (2, 0, 1))


# speedup vs baseline: 2.3301x; 1.4196x over previous
import functools

import jax
import jax.numpy as jnp
from jax import lax
from jax.experimental import pallas as pl
from jax.experimental.pallas import tpu as pltpu
from jax.experimental.pallas import tpu_sc as plsc

_NC = 2
_NS = 16
_NW = _NC * _NS
_L = 16

_BAND = 128
_WIDE = 128


@functools.lru_cache(maxsize=None)
def _make_gather(T, BATCH, V, D):
    mesh = plsc.VectorSubcoreMesh(core_axis_name="c", subcore_axis_name="s")

    @functools.partial(
        pl.kernel,
        out_type=jax.ShapeDtypeStruct((T, D, BATCH), jnp.float32),
        mesh=mesh,
        scratch_types=[
            pltpu.VMEM((2, 8, _BAND), jnp.int32),
            pltpu.VMEM((4, _BAND, _WIDE), jnp.float32),
            pltpu.VMEM((2, D, _BAND), jnp.float32),
            pltpu.SemaphoreType.DMA,
            pltpu.SemaphoreType.DMA,
            pltpu.SemaphoreType.DMA,
            pltpu.SemaphoreType.DMA,
            pltpu.SemaphoreType.DMA,
            pltpu.SemaphoreType.DMA,
        ],
        compiler_params=pltpu.CompilerParams(
            use_tc_tiling_on_sc=True, needs_layout_passes=False),
    )
    def k(table_hbm, idx_hbm, out_hbm, idx_v, rows_v, trans_v,
          gsem0, gsem1, gsem2, gsem3, ssem0, ssem1):
        wid = lax.axis_index("s") * _NC + lax.axis_index("c")
        bcol = pl.multiple_of(wid * _BAND, _BAND)

        iota = lax.iota(jnp.int32, _L)
        zero = iota - iota
        gsems = (gsem0, gsem1, gsem2, gsem3)
        ssems = (ssem0, ssem1)

        def load_octet(u):
            row = pl.multiple_of((u // 8) * 8, 8)
            ob = lax.rem(u // 8, 2)
            pltpu.sync_copy(idx_hbm.at[pl.ds(row, 8), pl.ds(bcol, _BAND)],
                            idx_v.at[ob])

        def idx_ref(u):
            ob = lax.rem(u // 8, 2)
            ts = lax.rem(u, 8)
            return idx_v.at[ob, ts]

        def gather_start(u, gbuf):
            pltpu.async_copy(table_hbm.at[idx_ref(u)], rows_v.at[gbuf],
                             gsems[gbuf])

        def gather_wait(u, gbuf):
            pltpu.make_async_copy(table_hbm.at[idx_ref(u)], rows_v.at[gbuf],
                                  gsems[gbuf]).wait()

        def out_slice(u):
            return out_hbm.at[u, :, pl.ds(bcol, _BAND)]

        def store_start(u, tbuf, par):
            pltpu.async_copy(trans_v.at[tbuf], out_slice(u), ssems[par])

        def store_wait(u, tbuf, par):
            pltpu.make_async_copy(trans_v.at[tbuf], out_slice(0),
                                  ssems[par]).wait()

        colpat = [(iota + kk) & 15 for kk in range(_L)]

        def transpose(tbuf, gbuf):
            rv = rows_v.at[gbuf]
            tv = trans_v.at[tbuf]
            for g in range(_BAND // _L):
                jb = g * _L
                rowv = iota + jb

                @plsc.parallel_loop(0, D // _L, 1, unroll=2)
                def _(cb):
                    c0 = cb * _L
                    for kk in range(_L):
                        cols = colpat[kk] + c0
                        val = plsc.load_gather(rv, [rowv, cols])
                        plsc.store_scatter(tv, [cols, rowv], val)

        load_octet(0)
        gather_start(0, 0)
        gather_start(1, 1)

        def step(u, s):
            par = s % 2
            nxt = u + 2

            def launch_next():
                pl.when(lax.rem(nxt, 8) == 0)(lambda: load_octet(nxt))
                gather_start(nxt, (s + 2) % 4)

            pl.when(nxt < T)(launch_next)
            pl.when(u >= 2)(lambda: store_wait(u - 2, par, par))
            gather_wait(u, s)
            transpose(par, s)
            store_start(u, par, par)

        def quad_body(p, carry):
            for s in range(4):
                step(4 * p + s, s)
            return carry

        lax.fori_loop(0, T // 4, quad_body, 0)
        store_wait(T - 2, 0, 0)
        store_wait(T - 1, 1, 1)

    return k


def kernel(words, weight):
    BATCH, T = words.shape
    V, D = weight.shape
    wpad = jnp.pad(weight, ((0, 0), (0, _WIDE - D)))
    idx_t = words.T.astype(jnp.int32)
    out3 = _make_gather(T, BATCH, V, D)(wpad, idx_t)
    return jnp.transpose(out3, (2, 0, 1))


# speedup vs baseline: 2.9135x; 1.2503x over previous
import functools

import jax
import jax.numpy as jnp
from jax import lax
from jax.experimental import pallas as pl
from jax.experimental.pallas import tpu as pltpu
from jax.experimental.pallas import tpu_sc as plsc

_NC = 2
_NS = 16
_NW = _NC * _NS
_L = 16

_BAND = 128
_WIDE = 128


@functools.lru_cache(maxsize=None)
def _make_transpose(V, D):
    n_full = V // _WIDE
    tail_rows = V - n_full * _WIDE

    mesh = plsc.VectorSubcoreMesh(core_axis_name="c", subcore_axis_name="s")

    @functools.partial(
        pl.kernel,
        out_type=jax.ShapeDtypeStruct((V, _WIDE), jnp.float32),
        mesh=mesh,
        scratch_types=[
            pltpu.VMEM((2, D, _WIDE), jnp.float32),
            pltpu.VMEM((2, _WIDE, _WIDE), jnp.float32),
            pltpu.SemaphoreType.DMA,
            pltpu.SemaphoreType.DMA,
            pltpu.SemaphoreType.DMA,
            pltpu.SemaphoreType.DMA,
        ],
        compiler_params=pltpu.CompilerParams(
            use_tc_tiling_on_sc=True, needs_layout_passes=False),
    )
    def k(wt_hbm, tail_hbm, out_hbm, in_v, tr_v, lsem0, lsem1, ssem0, ssem1):
        wid = lax.axis_index("s") * _NC + lax.axis_index("c")
        nc = (n_full // _NW) + jnp.where(wid < n_full % _NW, 1, 0)

        iota = lax.iota(jnp.int32, _L)
        colpat = [(iota + kk) & (_L - 1) for kk in range(_L)]
        lsems = (lsem0, lsem1)
        ssems = (ssem0, ssem1)

        def chunk(i):
            return pl.multiple_of((wid + _NW * i) * _WIDE, _WIDE)

        def load_start(i, buf):
            pltpu.async_copy(wt_hbm.at[:, pl.ds(chunk(i), _WIDE)],
                             in_v.at[buf], lsems[buf])

        def load_wait(i, buf):
            pltpu.make_async_copy(wt_hbm.at[:, pl.ds(chunk(i), _WIDE)],
                                  in_v.at[buf], lsems[buf]).wait()

        def store_start(i, buf):
            pltpu.async_copy(tr_v.at[buf], out_hbm.at[pl.ds(chunk(i), _WIDE)],
                             ssems[buf])

        def store_wait(buf):
            pltpu.make_async_copy(tr_v.at[buf],
                                  out_hbm.at[pl.ds(0, _WIDE)],
                                  ssems[buf]).wait()

        def transpose(buf):
            sv = in_v.at[buf]
            dv = tr_v.at[buf]
            for g in range(_WIDE // _L):
                jb = g * _L
                rowv = iota + jb

                @plsc.parallel_loop(0, D // _L, 1, unroll=2)
                def _(cb):
                    c0 = cb * _L
                    for kk in range(_L):
                        cols = colpat[kk] + c0
                        val = plsc.load_gather(sv, [cols, rowv])
                        plsc.store_scatter(dv, [rowv, cols], val)

        load_start(0, 0)
        pl.when(wid == _NW - 1)(lambda: pltpu.sync_copy(
            tail_hbm, out_hbm.at[pl.ds(n_full * _WIDE, tail_rows)]))

        def step(i, s):
            def work():
                load_wait(i, s)
                pl.when(i + 1 < nc)(lambda: load_start(i + 1, 1 - s))
                pl.when(i >= 2)(lambda: store_wait(s))
                transpose(s)
                store_start(i, s)

            pl.when(i < nc)(work)

        def pair_body(p, carry):
            step(2 * p, 0)
            step(2 * p + 1, 1)
            return carry

        n_pairs = (n_full // _NW + 2) // 2
        lax.fori_loop(0, n_pairs, pair_body, 0)
        store_wait(0)
        store_wait(1)

    return k


@functools.lru_cache(maxsize=None)
def _make_gather(T, BATCH, V, D):
    mesh = plsc.VectorSubcoreMesh(core_axis_name="c", subcore_axis_name="s")

    @functools.partial(
        pl.kernel,
        out_type=jax.ShapeDtypeStruct((T, D, BATCH), jnp.float32),
        mesh=mesh,
        scratch_types=[
            pltpu.VMEM((2, 8, _BAND), jnp.int32),
            pltpu.VMEM((4, _BAND, _WIDE), jnp.float32),
            pltpu.VMEM((2, D, _BAND), jnp.float32),
            pltpu.SemaphoreType.DMA,
            pltpu.SemaphoreType.DMA,
            pltpu.SemaphoreType.DMA,
            pltpu.SemaphoreType.DMA,
            pltpu.SemaphoreType.DMA,
            pltpu.SemaphoreType.DMA,
        ],
        compiler_params=pltpu.CompilerParams(
            use_tc_tiling_on_sc=True, needs_layout_passes=False),
    )
    def k(table_hbm, idx_hbm, out_hbm, idx_v, rows_v, trans_v,
          gsem0, gsem1, gsem2, gsem3, ssem0, ssem1):
        wid = lax.axis_index("s") * _NC + lax.axis_index("c")
        bcol = pl.multiple_of(wid * _BAND, _BAND)

        iota = lax.iota(jnp.int32, _L)
        zero = iota - iota
        gsems = (gsem0, gsem1, gsem2, gsem3)
        ssems = (ssem0, ssem1)

        def load_octet(u):
            row = pl.multiple_of((u // 8) * 8, 8)
            ob = lax.rem(u // 8, 2)
            pltpu.sync_copy(idx_hbm.at[pl.ds(row, 8), pl.ds(bcol, _BAND)],
                            idx_v.at[ob])

        def idx_ref(u):
            ob = lax.rem(u // 8, 2)
            ts = lax.rem(u, 8)
            return idx_v.at[ob, ts]

        def gather_start(u, gbuf):
            pltpu.async_copy(table_hbm.at[idx_ref(u)], rows_v.at[gbuf],
                             gsems[gbuf])

        def gather_wait(u, gbuf):
            pltpu.make_async_copy(table_hbm.at[idx_ref(u)], rows_v.at[gbuf],
                                  gsems[gbuf]).wait()

        def out_slice(u):
            return out_hbm.at[u, :, pl.ds(bcol, _BAND)]

        def store_start(u, tbuf, par):
            pltpu.async_copy(trans_v.at[tbuf], out_slice(u), ssems[par])

        def store_wait(u, tbuf, par):
            pltpu.make_async_copy(trans_v.at[tbuf], out_slice(0),
                                  ssems[par]).wait()

        colpat = [(iota + kk) & 15 for kk in range(_L)]

        def transpose(tbuf, gbuf):
            rv = rows_v.at[gbuf]
            tv = trans_v.at[tbuf]
            for g in range(_BAND // _L):
                jb = g * _L
                rowv = iota + jb

                @plsc.parallel_loop(0, D // _L, 1, unroll=2)
                def _(cb):
                    c0 = cb * _L
                    for kk in range(_L):
                        cols = colpat[kk] + c0
                        val = plsc.load_gather(rv, [rowv, cols])
                        plsc.store_scatter(tv, [cols, rowv], val)

        load_octet(0)
        gather_start(0, 0)
        gather_start(1, 1)

        def step(u, s):
            par = s % 2
            nxt = u + 2

            def launch_next():
                pl.when(lax.rem(nxt, 8) == 0)(lambda: load_octet(nxt))
                gather_start(nxt, (s + 2) % 4)

            pl.when(nxt < T)(launch_next)
            pl.when(u >= 2)(lambda: store_wait(u - 2, par, par))
            gather_wait(u, s)
            transpose(par, s)
            store_start(u, par, par)

        def quad_body(p, carry):
            for s in range(4):
                step(4 * p + s, s)
            return carry

        lax.fori_loop(0, T // 4, quad_body, 0)
        store_wait(T - 2, 0, 0)
        store_wait(T - 1, 1, 1)

    return k


def kernel(words, weight):
    BATCH, T = words.shape
    V, D = weight.shape
    n_full = V // _WIDE
    tail = jnp.pad(weight[n_full * _WIDE:, :], ((0, 0), (0, _WIDE - D)))
    wpad = _make_transpose(V, D)(weight.T, tail)
    idx_t = words.T.astype(jnp.int32)
    out3 = _make_gather(T, BATCH, V, D)(wpad, idx_t)
    return jnp.transpose(out3, (2, 0, 1))
